# baseline (speedup 1.0000x reference)
import functools

import jax
import jax.numpy as jnp
from jax import lax
from jax.experimental import pallas as pl
from jax.experimental.pallas import tpu as pltpu
from jax.experimental.pallas import tpu_sc as plsc

_B = 16384
_D = 64
_NC = 2
_NS = 16
_NW = _NC * _NS
_TOT = 2 * _B
_PER_W = _TOT // _NW
_CHUNK = 128
_NCHUNK = _PER_W // _CHUNK
_EPS = 1e-5


def _sc_gather(table, idx2d):
    mesh = plsc.VectorSubcoreMesh(core_axis_name="c", subcore_axis_name="s")

    @functools.partial(
        pl.kernel,
        mesh=mesh,
        out_type=jax.ShapeDtypeStruct((_TOT, _D // 2), jnp.int32),
        scratch_types=[
            pltpu.VMEM((_NCHUNK, _CHUNK), jnp.int32),
            pltpu.VMEM((_PER_W, _D // 2), jnp.int32),
            pltpu.SemaphoreType.DMA,
        ],
        compiler_params=pltpu.CompilerParams(use_tc_tiling_on_sc=False),
    )
    def k(table_hbm, idx_hbm, out_hbm, idx_v, rows_v, sem):
        wid = lax.axis_index("s") * _NC + lax.axis_index("c")
        pltpu.sync_copy(idx_hbm.at[pl.ds(wid * _NCHUNK, _NCHUNK)], idx_v)
        copies = []
        for j in range(_NCHUNK):
            copies.append(
                pltpu.async_copy(
                    table_hbm.at[idx_v.at[j]],
                    rows_v.at[pl.ds(j * _CHUNK, _CHUNK)],
                    sem,
                )
            )
        for c in copies:
            c.wait()
        pltpu.sync_copy(rows_v, out_hbm.at[pl.ds(wid * _PER_W, _PER_W)])

    return k(table, idx2d)


def _tc_body(e_ref, g1_ref, b1_ref, g2_ref, b2_ref, out_ref):
    e1 = e_ref[0:_B, :].astype(jnp.float32)
    e2 = e_ref[_B:, :].astype(jnp.float32)
    m1 = jnp.mean(e1, axis=0, keepdims=True)
    m2 = jnp.mean(e2, axis=0, keepdims=True)
    d1 = e1 - m1
    d2 = e2 - m2
    v1 = jnp.mean(d1 * d1, axis=0, keepdims=True)
    v2 = jnp.mean(d2 * d2, axis=0, keepdims=True)
    a1 = g1_ref[...] * lax.rsqrt(v1 + _EPS)
    a2 = g2_ref[...] * lax.rsqrt(v2 + _EPS)
    n1 = d1 * a1 + b1_ref[...]
    n2 = d2 * a2 + b2_ref[...]
    ones = jnp.ones((_D,), dtype=jnp.float32)
    s = jax.lax.dot_general(
        ones, n1 * n2, (((0,), (1,)), ((), ())),
        preferred_element_type=jnp.float32)
    out_ref[...] = jax.nn.sigmoid(s)


def _tc_compute(e, g1, b1, g2, b2):
    return pl.pallas_call(
        _tc_body,
        out_shape=jax.ShapeDtypeStruct((_B,), jnp.float32),
    )(e, g1, b1, g2, b2)


def kernel(idx, table, gamma1, beta1, gamma2, beta2):
    idx2d = idx.T.reshape(_NW * _NCHUNK, _CHUNK).astype(jnp.int32)
    tab16 = table.astype(jnp.bfloat16).reshape(-1, _D // 2, 2)
    tab32 = jax.lax.bitcast_convert_type(tab16, jnp.int32)
    e32 = _sc_gather(tab32, idx2d)
    e = jax.lax.bitcast_convert_type(e32, jnp.bfloat16).reshape(_TOT, _D)
    return _tc_compute(
        e,
        gamma1.reshape(1, _D), beta1.reshape(1, _D),
        gamma2.reshape(1, _D), beta2.reshape(1, _D),
    )


# speedup vs baseline: 5.5692x; 5.5692x over previous
import functools

import jax
import jax.numpy as jnp
from jax import lax
from jax.experimental import pallas as pl
from jax.experimental.pallas import tpu as pltpu
from jax.experimental.pallas import tpu_sc as plsc

_B = 16384
_V = 1000000
_D = 64
_VBLK = 4096
_NBLK = 123
_PP = _VBLK * _NBLK
_NC = 2
_NS = 16
_NW = _NC * _NS
_TOT = 2 * _B
_PER_W = _TOT // _NW
_CHUNK = 128
_HB = 512
_EPS = 1e-5


def _tc_pack(table_t):
    def body(x1_ref, x2_ref, o_ref):
        o_ref[...] = jnp.concatenate(
            [jnp.transpose(x1_ref[...]), jnp.transpose(x2_ref[...])], axis=1
        )

    return pl.pallas_call(
        body,
        grid=(_NBLK,),
        in_specs=[
            pl.BlockSpec((_D, _VBLK), lambda i: (0, i)),
            pl.BlockSpec(
                (_D, _VBLK),
                lambda i: (0, jnp.minimum(i + _NBLK, (_V - 1) // _VBLK)),
            ),
        ],
        out_specs=pl.BlockSpec((_VBLK, 2 * _D), lambda i: (i, 0)),
        out_shape=jax.ShapeDtypeStruct((_PP, 2 * _D), jnp.float32),
    )(table_t, table_t)


def _sc_gather(packed, p4d):
    mesh = plsc.VectorSubcoreMesh(core_axis_name="c", subcore_axis_name="s")

    @functools.partial(
        pl.kernel,
        mesh=mesh,
        out_type=jax.ShapeDtypeStruct((_TOT, 2 * _D), jnp.float32),
        scratch_types=[
            pltpu.VMEM((_PER_W // _CHUNK, 1, _CHUNK), jnp.int32),
            pltpu.VMEM((_HB, 2 * _D), jnp.float32),
            pltpu.SemaphoreType.DMA,
        ],
        compiler_params=pltpu.CompilerParams(use_tc_tiling_on_sc=True),
    )
    def k(tab_hbm, idx_hbm, out_hbm, idx_v, rows_v, sem):
        wid = lax.axis_index("s") * _NC + lax.axis_index("c")
        base = wid * _PER_W
        pltpu.sync_copy(idx_hbm.at[wid], idx_v)
        for b in range(_PER_W // _HB):
            copies = []
            for j in range(_HB // _CHUNK):
                copies.append(
                    pltpu.async_copy(
                        tab_hbm.at[idx_v.at[b * (_HB // _CHUNK) + j, 0]],
                        rows_v.at[pl.ds(j * _CHUNK, _CHUNK)],
                        sem,
                    )
                )
            for c in copies:
                c.wait()
            pltpu.sync_copy(
                rows_v, out_hbm.at[pl.ds(base + b * _HB, _HB)]
            )

    return k(packed, p4d)


def _tc_body(e_ref, m_ref, g1_ref, b1_ref, g2_ref, b2_ref, out_ref):
    m = m_ref[...]
    e = jnp.where(m > 0.5, e_ref[:, _D:], e_ref[:, 0:_D])
    e1 = e[0:_B, :]
    e2 = e[_B:, :]
    m1 = jnp.mean(e1, axis=0, keepdims=True)
    m2 = jnp.mean(e2, axis=0, keepdims=True)
    d1 = e1 - m1
    d2 = e2 - m2
    v1 = jnp.mean(d1 * d1, axis=0, keepdims=True)
    v2 = jnp.mean(d2 * d2, axis=0, keepdims=True)
    a1 = g1_ref[...] * lax.rsqrt(v1 + _EPS)
    a2 = g2_ref[...] * lax.rsqrt(v2 + _EPS)
    n1 = d1 * a1 + b1_ref[...]
    n2 = d2 * a2 + b2_ref[...]
    ones = jnp.ones((_D,), dtype=jnp.float32)
    s = jax.lax.dot_general(
        ones, n1 * n2, (((0,), (1,)), ((), ())),
        preferred_element_type=jnp.float32)
    out_ref[...] = jax.nn.sigmoid(s)


def _tc_compute(e, m8, g1, b1, g2, b2):
    return pl.pallas_call(
        _tc_body,
        out_shape=jax.ShapeDtypeStruct((_B,), jnp.float32),
    )(e, m8, g1, b1, g2, b2)


def kernel(idx, table, gamma1, beta1, gamma2, beta2):
    flat = idx.T.reshape(_TOT).astype(jnp.int32)
    high = flat >= _PP
    p1d = jnp.where(high, flat - _PP, flat)
    m8 = jnp.broadcast_to(
        high.astype(jnp.float32).reshape(_TOT, 1), (_TOT, _D)
    )
    packed = _tc_pack(table.T)
    e = _sc_gather(packed, p1d.reshape(_NW, _PER_W // _CHUNK, 1, _CHUNK))
    return _tc_compute(
        e, m8,
        gamma1.reshape(1, _D), beta1.reshape(1, _D),
        gamma2.reshape(1, _D), beta2.reshape(1, _D),
    )


# speedup vs baseline: 6.1095x; 1.0970x over previous
import functools

import jax
import jax.numpy as jnp
from jax import lax
from jax.experimental import pallas as pl
from jax.experimental.pallas import tpu as pltpu
from jax.experimental.pallas import tpu_sc as plsc

_B = 16384
_V = 1000000
_D = 64
_VBLK = 8192
_NBLK = 62
_PP = _VBLK * _NBLK
_NC = 2
_NS = 16
_NW = _NC * _NS
_TOT = 2 * _B
_PER_W = _TOT // _NW
_CHUNK = 128
_HB = 512
_EPS = 1e-5


def _tc_pack(table_t):
    def body(x1_ref, x2_ref, o_ref):
        o_ref[...] = jnp.concatenate(
            [jnp.transpose(x1_ref[...]), jnp.transpose(x2_ref[...])], axis=1
        )

    return pl.pallas_call(
        body,
        grid=(_NBLK,),
        in_specs=[
            pl.BlockSpec((_D, _VBLK), lambda i: (0, i)),
            pl.BlockSpec(
                (_D, _VBLK),
                lambda i: (0, jnp.minimum(i + _NBLK, (_V - 1) // _VBLK)),
            ),
        ],
        out_specs=pl.BlockSpec((_VBLK, 2 * _D), lambda i: (i, 0)),
        out_shape=jax.ShapeDtypeStruct((_PP, 2 * _D), jnp.float32),
    )(table_t, table_t)


def _sc_gather(packed, p4d):
    mesh = plsc.VectorSubcoreMesh(core_axis_name="c", subcore_axis_name="s")

    @functools.partial(
        pl.kernel,
        mesh=mesh,
        out_type=jax.ShapeDtypeStruct((_TOT, 2 * _D), jnp.float32),
        scratch_types=[
            pltpu.VMEM((_PER_W // _CHUNK, 1, _CHUNK), jnp.int32),
            pltpu.VMEM((_HB, 2 * _D), jnp.float32),
            pltpu.SemaphoreType.DMA,
        ],
        compiler_params=pltpu.CompilerParams(use_tc_tiling_on_sc=True),
    )
    def k(tab_hbm, idx_hbm, out_hbm, idx_v, rows_v, sem):
        wid = lax.axis_index("s") * _NC + lax.axis_index("c")
        base = wid * _PER_W
        pltpu.sync_copy(idx_hbm.at[wid], idx_v)
        for b in range(_PER_W // _HB):
            copies = []
            for j in range(_HB // _CHUNK):
                copies.append(
                    pltpu.async_copy(
                        tab_hbm.at[idx_v.at[b * (_HB // _CHUNK) + j, 0]],
                        rows_v.at[pl.ds(j * _CHUNK, _CHUNK)],
                        sem,
                    )
                )
            for c in copies:
                c.wait()
            pltpu.sync_copy(
                rows_v, out_hbm.at[pl.ds(base + b * _HB, _HB)]
            )

    return k(packed, p4d)


def _tc_body(e_ref, m_ref, g1_ref, b1_ref, g2_ref, b2_ref, out_ref):
    m = m_ref[...] > 0.5
    e = jnp.where(m, e_ref[:, _D:], e_ref[:, 0:_D])
    e1 = e[0:_B, :]
    e2 = e[_B:, :]
    m1 = jnp.mean(e1, axis=0, keepdims=True)
    m2 = jnp.mean(e2, axis=0, keepdims=True)
    d1 = e1 - m1
    d2 = e2 - m2
    v1 = jnp.mean(d1 * d1, axis=0, keepdims=True)
    v2 = jnp.mean(d2 * d2, axis=0, keepdims=True)
    a1 = g1_ref[...] * lax.rsqrt(v1 + _EPS)
    a2 = g2_ref[...] * lax.rsqrt(v2 + _EPS)
    n1 = d1 * a1 + b1_ref[...]
    n2 = d2 * a2 + b2_ref[...]
    ones = jnp.ones((_D,), dtype=jnp.float32)
    s = jax.lax.dot_general(
        ones, n1 * n2, (((0,), (1,)), ((), ())),
        preferred_element_type=jnp.float32)
    out_ref[...] = jax.nn.sigmoid(s)


def _tc_compute(e, m8, g1, b1, g2, b2):
    return pl.pallas_call(
        _tc_body,
        out_shape=jax.ShapeDtypeStruct((_B,), jnp.float32),
    )(e, m8, g1, b1, g2, b2)


def kernel(idx, table, gamma1, beta1, gamma2, beta2):
    flat = idx.T.reshape(_TOT).astype(jnp.int32)
    high = flat >= _PP
    p1d = jnp.where(high, flat - _PP, flat)
    m8 = jnp.broadcast_to(
        high.astype(jnp.bfloat16).reshape(_TOT, 1), (_TOT, _D)
    )
    packed = _tc_pack(table.T)
    e = _sc_gather(packed, p1d.reshape(_NW, _PER_W // _CHUNK, 1, _CHUNK))
    return _tc_compute(
        e, m8,
        gamma1.reshape(1, _D), beta1.reshape(1, _D),
        gamma2.reshape(1, _D), beta2.reshape(1, _D),
    )


# speedup vs baseline: 6.4323x; 1.0528x over previous
import functools

import jax
import jax.numpy as jnp
from jax import lax
from jax.experimental import pallas as pl
from jax.experimental.pallas import tpu as pltpu
from jax.experimental.pallas import tpu_sc as plsc

_B = 16384
_V = 1000000
_D = 64
_VBLK = 16384
_NBLK = 31
_PP = _VBLK * _NBLK
_NC = 2
_NS = 16
_NW = _NC * _NS
_TOT = 2 * _B
_PER_W = _TOT // _NW
_CHUNK = 128
_HB = 512
_EPS = 1e-5


def _tc_pack(table_t):
    def body(x1_ref, x2_ref, o_ref):
        o_ref[...] = jnp.concatenate(
            [jnp.transpose(x1_ref[...]), jnp.transpose(x2_ref[...])], axis=1
        )

    return pl.pallas_call(
        body,
        grid=(_NBLK,),
        in_specs=[
            pl.BlockSpec((_D, _VBLK), lambda i: (0, i)),
            pl.BlockSpec(
                (_D, _VBLK),
                lambda i: (0, jnp.minimum(i + _NBLK, (_V - 1) // _VBLK)),
            ),
        ],
        out_specs=pl.BlockSpec((_VBLK, 2 * _D), lambda i: (i, 0)),
        out_shape=jax.ShapeDtypeStruct((_PP, 2 * _D), jnp.float32),
    )(table_t, table_t)


def _sc_gather(packed, p4d):
    mesh = plsc.VectorSubcoreMesh(core_axis_name="c", subcore_axis_name="s")

    @functools.partial(
        pl.kernel,
        mesh=mesh,
        out_type=jax.ShapeDtypeStruct((_TOT, 2 * _D), jnp.float32),
        scratch_types=[
            pltpu.VMEM((_PER_W // _CHUNK, 1, _CHUNK), jnp.int32),
            pltpu.VMEM((_HB, 2 * _D), jnp.float32),
            pltpu.SemaphoreType.DMA,
        ],
        compiler_params=pltpu.CompilerParams(use_tc_tiling_on_sc=True),
    )
    def k(tab_hbm, idx_hbm, out_hbm, idx_v, rows_v, sem):
        wid = lax.axis_index("s") * _NC + lax.axis_index("c")
        base = wid * _PER_W
        pltpu.sync_copy(idx_hbm.at[wid], idx_v)
        for b in range(_PER_W // _HB):
            copies = []
            for j in range(_HB // _CHUNK):
                copies.append(
                    pltpu.async_copy(
                        tab_hbm.at[idx_v.at[b * (_HB // _CHUNK) + j, 0]],
                        rows_v.at[pl.ds(j * _CHUNK, _CHUNK)],
                        sem,
                    )
                )
            for c in copies:
                c.wait()
            pltpu.sync_copy(
                rows_v, out_hbm.at[pl.ds(base + b * _HB, _HB)]
            )

    return k(packed, p4d)


def _tc_body(e_ref, m_ref, g1_ref, b1_ref, g2_ref, b2_ref, out_ref):
    m = m_ref[...] > 0.5
    e = jnp.where(m, e_ref[:, _D:], e_ref[:, 0:_D])
    e1 = e[0:_B, :]
    e2 = e[_B:, :]
    m1 = jnp.mean(e1, axis=0, keepdims=True)
    m2 = jnp.mean(e2, axis=0, keepdims=True)
    d1 = e1 - m1
    d2 = e2 - m2
    v1 = jnp.mean(d1 * d1, axis=0, keepdims=True)
    v2 = jnp.mean(d2 * d2, axis=0, keepdims=True)
    a1 = g1_ref[...] * lax.rsqrt(v1 + _EPS)
    a2 = g2_ref[...] * lax.rsqrt(v2 + _EPS)
    n1 = d1 * a1 + b1_ref[...]
    n2 = d2 * a2 + b2_ref[...]
    ones = jnp.ones((_D,), dtype=jnp.float32)
    s = jax.lax.dot_general(
        ones, n1 * n2, (((0,), (1,)), ((), ())),
        preferred_element_type=jnp.float32)
    out_ref[...] = jax.nn.sigmoid(s)


def _tc_compute(e, m8, g1, b1, g2, b2):
    return pl.pallas_call(
        _tc_body,
        out_shape=jax.ShapeDtypeStruct((_B,), jnp.float32),
    )(e, m8, g1, b1, g2, b2)


def kernel(idx, table, gamma1, beta1, gamma2, beta2):
    flat = idx.T.reshape(_TOT).astype(jnp.int32)
    high = flat >= _PP
    p1d = jnp.where(high, flat - _PP, flat)
    m8 = jnp.broadcast_to(
        high.astype(jnp.bfloat16).reshape(_TOT, 1), (_TOT, _D)
    )
    packed = _tc_pack(table.T)
    e = _sc_gather(packed, p1d.reshape(_NW, _PER_W // _CHUNK, 1, _CHUNK))
    return _tc_compute(
        e, m8,
        gamma1.reshape(1, _D), beta1.reshape(1, _D),
        gamma2.reshape(1, _D), beta2.reshape(1, _D),
    )
